# Initial kernel scaffold; baseline (speedup 1.0000x reference)
import jax
import jax.numpy as jnp
from jax.experimental import pallas as pl


def kernel(x, edge_index, edge_weight, W1, b1, W2, b2):
    raise NotImplementedError("write your pallas kernel here")


# speedup vs baseline: 1.7917x; 1.7917x over previous
import functools

import jax
import jax.numpy as jnp
from jax import lax
from jax.experimental import pallas as pl
from jax.experimental.pallas import tpu as pltpu
from jax.experimental.pallas import tpu_sc as plsc

N = 10000
D = 128
DH = D // 2
E = 320000
NC = 2
NS = 16
L = 16
NW = NC * NS
CHUNK = 128
EPT = 10240
NCHUNK = EPT // CHUNK
E_PAD = NW * EPT
N_ACC = 10240
ROWS_PT = N_ACC // NS
RZ = 128

_mesh = plsc.VectorSubcoreMesh(core_axis_name="c", subcore_axis_name="s")


@functools.partial(
    pl.kernel,
    mesh=_mesh,
    out_type=jax.ShapeDtypeStruct((NC, 2, N_ACC, DH), jnp.float32),
    scratch_types=[
        pltpu.VMEM((NCHUNK, CHUNK), jnp.int32),
        pltpu.VMEM((NCHUNK, CHUNK), jnp.int32),
        pltpu.VMEM((NCHUNK, CHUNK), jnp.int32),
        pltpu.VMEM((NCHUNK, CHUNK), jnp.float32),
        pltpu.VMEM((CHUNK, DH), jnp.float32),
        pltpu.VMEM((RZ, DH), jnp.float32),
        pltpu.VMEM_SHARED((N_ACC, DH), jnp.float32),
        pltpu.SemaphoreType.DMA,
    ],
    compiler_params=pltpu.CompilerParams(use_tc_tiling_on_sc=False),
)
def _spmm_sc(support_hbm, srcA_hbm, srcB_hbm, dst_hbm, w_hbm, out_hbm,
             srcA_v, srcB_v, dst_v, w_v, rows_v, zbuf, acc, sem):
    c = lax.axis_index("c")
    s = lax.axis_index("s")
    wid = c * NS + s

    def _zb(i, carry):
        for j in range(DH // L):
            zbuf[i, pl.ds(j * L, L)] = jnp.zeros((L,), jnp.float32)
        return carry
    lax.fori_loop(0, RZ, _zb, 0)

    pltpu.sync_copy(srcA_hbm.at[pl.ds(wid * NCHUNK, NCHUNK)], srcA_v)
    pltpu.sync_copy(srcB_hbm.at[pl.ds(wid * NCHUNK, NCHUNK)], srcB_v)
    pltpu.sync_copy(dst_hbm.at[pl.ds(wid * NCHUNK, NCHUNK)], dst_v)
    pltpu.sync_copy(w_hbm.at[pl.ds(wid * NCHUNK, NCHUNK)], w_v)

    for h in range(2):
        src_v = srcA_v if h == 0 else srcB_v
        for q in range(ROWS_PT // RZ):
            pltpu.sync_copy(zbuf, acc.at[pl.ds(s * ROWS_PT + q * RZ, RZ)])
        plsc.subcore_barrier()

        def _chunk(k, carry):
            pltpu.async_copy(support_hbm.at[src_v.at[k]], rows_v, sem).wait()

            def _scale(g, cc):
                wvec = w_v[k, pl.ds(g * L, L)]
                for li in range(L):
                    e = g * L + li
                    w = wvec[li]
                    for j in range(DH // L):
                        sl = pl.ds(j * L, L)
                        rows_v[e, sl] = rows_v[e, sl] * w
                return cc
            lax.fori_loop(0, CHUNK // L, _scale, 0)

            pltpu.sync_copy(rows_v, acc.at[dst_v.at[k]], add=True)
            return carry
        lax.fori_loop(0, NCHUNK, _chunk, 0)

        plsc.subcore_barrier()
        pltpu.sync_copy(acc.at[pl.ds(s * ROWS_PT, ROWS_PT)],
                        out_hbm.at[c, h, pl.ds(s * ROWS_PT, ROWS_PT)])
        plsc.subcore_barrier()


_BR = 1000


def _mm1_body(x_ref, w_ref, o_ref):
    o_ref[...] = jnp.dot(x_ref[...], w_ref[...],
                         preferred_element_type=jnp.float32)


def _mm1(x, W):
    return pl.pallas_call(
        _mm1_body,
        grid=(N // _BR,),
        in_specs=[
            pl.BlockSpec((_BR, D), lambda k: (k, 0)),
            pl.BlockSpec((D, D), lambda k: (0, 0)),
        ],
        out_specs=pl.BlockSpec((_BR, D), lambda k: (k, 0)),
        out_shape=jax.ShapeDtypeStruct((N, D), jnp.float32),
    )(x, W)


def _mm2_body(p_ref, b_ref, w_ref, o_ref):
    x0 = jnp.maximum(p_ref[0, 0] + p_ref[1, 0] + b_ref[0], 0.0)
    x1 = jnp.maximum(p_ref[0, 1] + p_ref[1, 1] + b_ref[1], 0.0)
    o_ref[...] = (
        jnp.dot(x0, w_ref[0], preferred_element_type=jnp.float32)
        + jnp.dot(x1, w_ref[1], preferred_element_type=jnp.float32)
    )


def _mm2(p, b_halves, W_halves):
    return pl.pallas_call(
        _mm2_body,
        grid=(N // _BR,),
        in_specs=[
            pl.BlockSpec((NC, 2, _BR, DH), lambda k: (0, 0, k, 0)),
            pl.BlockSpec((2, 1, DH), lambda k: (0, 0, 0)),
            pl.BlockSpec((2, DH, D), lambda k: (0, 0, 0)),
        ],
        out_specs=pl.BlockSpec((_BR, D), lambda k: (k, 0)),
        out_shape=jax.ShapeDtypeStruct((N, D), jnp.float32),
    )(p, b_halves, W_halves)


def _fin_body(p_ref, b_ref, o_ref):
    o_ref[:, 0, :] = p_ref[0, 0] + p_ref[1, 0] + b_ref[0]
    o_ref[:, 1, :] = p_ref[0, 1] + p_ref[1, 1] + b_ref[1]


def _fin(p, b_halves):
    return pl.pallas_call(
        _fin_body,
        grid=(N // _BR,),
        in_specs=[
            pl.BlockSpec((NC, 2, _BR, DH), lambda k: (0, 0, k, 0)),
            pl.BlockSpec((2, 1, DH), lambda k: (0, 0, 0)),
        ],
        out_specs=pl.BlockSpec((_BR, 2, DH), lambda k: (k, 0, 0)),
        out_shape=jax.ShapeDtypeStruct((N, 2, DH), jnp.float32),
    )(p, b_halves)


def kernel(x, edge_index, edge_weight, W1, b1, W2, b2):
    src = edge_index[0].astype(jnp.int32)
    dst = edge_index[1].astype(jnp.int32)
    w = edge_weight.astype(jnp.float32)
    pad = E_PAD - E
    srcA = jnp.pad(2 * src, (0, pad)).reshape(NW * NCHUNK, CHUNK)
    srcB = jnp.pad(2 * src + 1, (0, pad)).reshape(NW * NCHUNK, CHUNK)
    dst2 = jnp.pad(dst, (0, pad)).reshape(NW * NCHUNK, CHUNK)
    w2 = jnp.pad(w, (0, pad)).reshape(NW * NCHUNK, CHUNK)
    b1h = b1.reshape(2, 1, DH)
    b2h = b2.reshape(2, 1, DH)
    W2h = W2.reshape(2, DH, D)

    support1 = _mm1(x, W1).reshape(2 * N, DH)
    p1 = _spmm_sc(support1, srcA, srcB, dst2, w2)
    support2 = _mm2(p1, b1h, W2h).reshape(2 * N, DH)
    p2 = _spmm_sc(support2, srcA, srcB, dst2, w2)
    return _fin(p2, b2h).reshape(N, D)
